# baseline (speedup 1.0000x reference)
import functools

import jax
import jax.numpy as jnp
from jax import lax
from jax.experimental import pallas as pl
from jax.experimental.pallas import tpu as pltpu
from jax.experimental.pallas import tpu_sc as plsc

VOCAB = 100000
D = 128
HD = 128
B = 16384
H = 20

NC = 2
NS = 16
NW = NC * NS
L = 16

CHUNK = 4
IDX_PER_CHUNK = CHUNK * H
NBUF = 8
GROUP_ROWS = NBUF * CHUNK

_mesh = plsc.VectorSubcoreMesh(core_axis_name="c", subcore_axis_name="s")


def _make_sc_pool(nb):
    b_per_w = nb // NW
    nchunk = b_per_w // CHUNK

    scratch = (
        [pltpu.VMEM((b_per_w * H,), jnp.int32)]
        + [pltpu.VMEM((IDX_PER_CHUNK, D), jnp.float32)] * NBUF
        + [pltpu.VMEM((GROUP_ROWS, D), jnp.float32)] * 2
        + [pltpu.SemaphoreType.DMA] * NBUF
        + [pltpu.SemaphoreType.DMA] * 2
    )

    @functools.partial(
        pl.kernel,
        out_type=jax.ShapeDtypeStruct((nb, D), jnp.float32),
        mesh=_mesh,
        scratch_types=scratch,
    )
    def sc_pool(x_hbm, table_hbm, out_hbm, idx_v, *refs):
        rows = refs[:NBUF]
        pooled = refs[NBUF:NBUF + 2]
        gsems = refs[NBUF + 2:2 * NBUF + 2]
        osems = refs[2 * NBUF + 2:]

        wid = lax.axis_index("s") * NC + lax.axis_index("c")
        out_base = wid * b_per_w

        pltpu.sync_copy(x_hbm.at[pl.ds(out_base * H, b_per_w * H)], idx_v)

        def idx_slice(chunk):
            return idx_v.at[pl.ds(chunk * IDX_PER_CHUNK, IDX_PER_CHUNK)]

        def start(chunk, b):
            pltpu.async_copy(table_hbm.at[idx_slice(chunk)], rows[b], gsems[b])

        def wait(chunk, b):
            pltpu.make_async_copy(
                table_hbm.at[idx_slice(chunk)], rows[b], gsems[b]).wait()

        def out_slice(chunk):
            return out_hbm.at[pl.ds(out_base + chunk * CHUNK, GROUP_ROWS)]

        def reduce_chunk(rows_buf, pooled_buf, row_base):
            @pl.loop(0, D, step=L)
            def _(d):
                for c in range(CHUNK):
                    v = [rows_buf[c * H + h, pl.ds(d, L)] for h in range(H)]
                    while len(v) > 1:
                        nxt = [v[i] + v[i + 1] for i in range(0, len(v) - 1, 2)]
                        if len(v) % 2:
                            nxt.append(v[-1])
                        v = nxt
                    pooled_buf[row_base + c, pl.ds(d, L)] = v[0]

        for b in range(NBUF):
            start(b, b)

        @pl.loop(0, nchunk, step=2 * NBUF)
        def _(i):
            for half in range(2):
                pooled_b, osem = pooled[half], osems[half]

                @pl.when(i > 0)
                def _():
                    pltpu.make_async_copy(
                        pooled_b, out_slice(i + half * NBUF), osem).wait()

                for b in range(NBUF):
                    chunk = i + half * NBUF + b
                    wait(chunk, b)
                    reduce_chunk(rows[b], pooled_b, b * CHUNK)

                    nxt = chunk + NBUF

                    @pl.when(nxt < nchunk)
                    def _():
                        start(nxt, b)

                pltpu.async_copy(pooled_b, out_slice(i + half * NBUF), osem)

        for half in range(2):
            pltpu.make_async_copy(pooled[half], out_slice(0), osems[half]).wait()

    return sc_pool


BLK = 2048


def _head_body(pooled_ref, w1_ref, b1_ref, w2_ref, b2_ref, out_ref):
    p = pooled_ref[...]
    w1 = w1_ref[...] * (1.0 / H)
    h = jnp.dot(p, w1, preferred_element_type=jnp.float32) + b1_ref[...]
    h = jnp.maximum(h, 0.0)
    out = jnp.sum(h * w2_ref[...], axis=1) + b2_ref[0, 0]
    out_ref[...] = out


def _make_head(nb):
    return pl.pallas_call(
        _head_body,
        grid=(nb // BLK,),
        in_specs=[
            pl.BlockSpec((BLK, D), lambda i: (i, 0)),
            pl.BlockSpec((D, HD), lambda i: (0, 0)),
            pl.BlockSpec((1, HD), lambda i: (0, 0)),
            pl.BlockSpec((1, HD), lambda i: (0, 0)),
            pl.BlockSpec((1, 1), lambda i: (0, 0), memory_space=pltpu.SMEM),
        ],
        out_specs=pl.BlockSpec((BLK,), lambda i: (i,)),
        out_shape=jax.ShapeDtypeStruct((nb,), jnp.float32),
    )


NSPLIT = 4
NB = B // NSPLIT
_sc_pool_half = _make_sc_pool(NB)
_head_half = _make_head(NB)


def kernel(x, embed_table, W1, b1, W2, b2):
    b1r, w2r, b2r = b1.reshape(1, HD), W2.reshape(1, HD), b2.reshape(1, 1)
    outs = []
    for s in range(NSPLIT):
        x_s = x[s * NB:(s + 1) * NB].reshape(NB * H)
        pooled = _sc_pool_half(x_s, embed_table)
        outs.append(_head_half(pooled, W1, b1r, w2r, b2r))
    return jnp.concatenate(outs)


# speedup vs baseline: 1.1979x; 1.1979x over previous
import functools

import jax
import jax.numpy as jnp
from jax import lax
from jax.experimental import pallas as pl
from jax.experimental.pallas import tpu as pltpu
from jax.experimental.pallas import tpu_sc as plsc

VOCAB = 100000
D = 128
HD = 128
B = 16384
H = 20

NC = 2
NS = 16
NW = NC * NS
L = 16

CHUNK = 4
IDX_PER_CHUNK = CHUNK * H
NBUF = 8
GROUP_ROWS = NBUF * CHUNK

_mesh = plsc.VectorSubcoreMesh(core_axis_name="c", subcore_axis_name="s")


def _make_sc_pool(nb):
    b_per_w = nb // NW
    nchunk = b_per_w // CHUNK

    scratch = (
        [pltpu.VMEM((b_per_w * H,), jnp.int32)]
        + [pltpu.VMEM((IDX_PER_CHUNK, D), jnp.float32)] * NBUF
        + [pltpu.VMEM((GROUP_ROWS, D), jnp.float32)] * 2
        + [pltpu.SemaphoreType.DMA] * NBUF
        + [pltpu.SemaphoreType.DMA] * 2
    )

    @functools.partial(
        pl.kernel,
        out_type=jax.ShapeDtypeStruct((nb, D), jnp.float32),
        mesh=_mesh,
        scratch_types=scratch,
    )
    def sc_pool(x_hbm, table_hbm, out_hbm, idx_v, *refs):
        rows = refs[:NBUF]
        pooled = refs[NBUF:NBUF + 2]
        gsems = refs[NBUF + 2:2 * NBUF + 2]
        osems = refs[2 * NBUF + 2:]

        wid = lax.axis_index("s") * NC + lax.axis_index("c")
        out_base = wid * b_per_w

        pltpu.sync_copy(x_hbm.at[pl.ds(out_base * H, b_per_w * H)], idx_v)

        def idx_slice(chunk):
            return idx_v.at[pl.ds(chunk * IDX_PER_CHUNK, IDX_PER_CHUNK)]

        def start(chunk, b):
            pltpu.async_copy(table_hbm.at[idx_slice(chunk)], rows[b], gsems[b])

        def wait(chunk, b):
            pltpu.make_async_copy(
                table_hbm.at[idx_slice(chunk)], rows[b], gsems[b]).wait()

        def out_slice(chunk):
            return out_hbm.at[pl.ds(out_base + chunk * CHUNK, GROUP_ROWS)]

        def reduce_chunk(rows_buf, pooled_buf, row_base):
            @pl.loop(0, D, step=L)
            def _(d):
                for c in range(CHUNK):
                    v = [rows_buf[c * H + h, pl.ds(d, L)] for h in range(H)]
                    while len(v) > 1:
                        nxt = [v[i] + v[i + 1] for i in range(0, len(v) - 1, 2)]
                        if len(v) % 2:
                            nxt.append(v[-1])
                        v = nxt
                    pooled_buf[row_base + c, pl.ds(d, L)] = v[0]

        for b in range(NBUF):
            start(b, b)

        @pl.loop(0, nchunk, step=2 * NBUF)
        def _(i):
            for half in range(2):
                pooled_b, osem = pooled[half], osems[half]

                @pl.when(i > 0)
                def _():
                    pltpu.make_async_copy(
                        pooled_b, out_slice(i + half * NBUF), osem).wait()

                for b in range(NBUF):
                    chunk = i + half * NBUF + b
                    wait(chunk, b)
                    reduce_chunk(rows[b], pooled_b, b * CHUNK)

                    nxt = chunk + NBUF

                    @pl.when(nxt < nchunk)
                    def _():
                        start(nxt, b)

                pltpu.async_copy(pooled_b, out_slice(i + half * NBUF), osem)

        for half in range(2):
            pltpu.make_async_copy(pooled[half], out_slice(0), osems[half]).wait()

    return sc_pool


BLK = 2048


def _head_body(pooled_ref, w1_ref, b1_ref, w2_ref, b2_ref, out_ref):
    p = pooled_ref[...]
    w1 = w1_ref[...] * (1.0 / H)
    h = jnp.dot(p, w1, preferred_element_type=jnp.float32) + b1_ref[...]
    h = jnp.maximum(h, 0.0)
    out = jnp.sum(h * w2_ref[...], axis=1) + b2_ref[0, 0]
    out_ref[...] = out


def _make_head(nb):
    return pl.pallas_call(
        _head_body,
        grid=(nb // BLK,),
        in_specs=[
            pl.BlockSpec((BLK, D), lambda i: (i, 0)),
            pl.BlockSpec((D, HD), lambda i: (0, 0)),
            pl.BlockSpec((1, HD), lambda i: (0, 0)),
            pl.BlockSpec((1, HD), lambda i: (0, 0)),
            pl.BlockSpec((1, 1), lambda i: (0, 0), memory_space=pltpu.SMEM),
        ],
        out_specs=pl.BlockSpec((BLK,), lambda i: (i,)),
        out_shape=jax.ShapeDtypeStruct((nb,), jnp.float32),
    )


NSPLIT = 2
NB = B // NSPLIT
_sc_pool_half = _make_sc_pool(NB)
_head_half = _make_head(NB)


def kernel(x, embed_table, W1, b1, W2, b2):
    b1r, w2r, b2r = b1.reshape(1, HD), W2.reshape(1, HD), b2.reshape(1, 1)
    pooled = []
    for s in range(NSPLIT):
        x_s = x[s * NB:(s + 1) * NB].reshape(NB * H)
        pooled.append(_sc_pool_half(x_s, embed_table))
    outs = [_head_half(p, W1, b1r, w2r, b2r) for p in pooled]
    return jnp.concatenate(outs)


# speedup vs baseline: 1.2640x; 1.0552x over previous
import dataclasses
import functools

import jax
import jax.numpy as jnp
from jax import lax
from jax.experimental import pallas as pl
from jax.experimental.pallas import tpu as pltpu
from jax.experimental.pallas import tpu_sc as plsc

VOCAB = 100000
D = 128
HD = 128
B = 16384
H = 20

NC = 2
NS = 16
NW = NC * NS
L = 16

B_PER_W = B // NW
CHUNK = 4
IDX_PER_CHUNK = CHUNK * H
NCHUNK = B_PER_W // CHUNK
NBUF = 8
GROUP_ROWS = NBUF * CHUNK

_mesh = plsc.VectorSubcoreMesh(core_axis_name="c", subcore_axis_name="s")

_cp = pltpu.CompilerParams()
if "needs_layout_passes" in pltpu.CompilerParams.__dataclass_fields__:
    _cp = dataclasses.replace(_cp, needs_layout_passes=False)

_scratch = (
    [pltpu.VMEM((B_PER_W // 4, H), jnp.int32)]
    + [pltpu.VMEM((B_PER_W * H,), jnp.int32)]
    + [pltpu.VMEM((IDX_PER_CHUNK, D), jnp.float32)] * NBUF
    + [pltpu.VMEM((GROUP_ROWS, D), jnp.float32)] * 2
    + [pltpu.SemaphoreType.DMA] * NBUF
    + [pltpu.SemaphoreType.DMA] * 2
)


@functools.partial(
    pl.kernel,
    out_type=jax.ShapeDtypeStruct((B, D), jnp.float32),
    mesh=_mesh,
    scratch_types=_scratch,
    compiler_params=_cp,
)
def _sc_pool(x_hbm, table_hbm, out_hbm, idx2d_v, idx_v, *refs):
    rows = refs[:NBUF]
    pooled = refs[NBUF:NBUF + 2]
    gsems = refs[NBUF + 2:2 * NBUF + 2]
    osems = refs[2 * NBUF + 2:]

    wid = lax.axis_index("s") * NC + lax.axis_index("c")
    out_base = wid * B_PER_W

    lane = lax.broadcasted_iota(jnp.int32, (L,), 0)
    patt = []
    for k in range(5):
        pos = lane + (k * L)
        patt.append((pos // H, pos % H))

    PIECE = B_PER_W // 4
    for piece in range(4):
        pltpu.sync_copy(x_hbm.at[pl.ds(out_base + piece * PIECE, PIECE)], idx2d_v)

        @pl.loop(0, PIECE // CHUNK)
        def _(sg, piece=piece):
            row_off = sg * CHUNK
            for k in range(5):
                r, c = patt[k]
                vals = plsc.load_gather(idx2d_v, [r + row_off, c])
                idx_v[pl.ds(piece * PIECE * H + sg * IDX_PER_CHUNK + k * L, L)] = vals

    def idx_slice(chunk):
        return idx_v.at[pl.ds(chunk * IDX_PER_CHUNK, IDX_PER_CHUNK)]

    def start(chunk, b):
        pltpu.async_copy(table_hbm.at[idx_slice(chunk)], rows[b], gsems[b])

    def wait(chunk, b):
        pltpu.make_async_copy(table_hbm.at[idx_slice(chunk)], rows[b], gsems[b]).wait()

    def out_slice(chunk):
        return out_hbm.at[pl.ds(out_base + chunk * CHUNK, GROUP_ROWS)]

    def reduce_chunk(rows_buf, pooled_buf, row_base):
        @pl.loop(0, D, step=L)
        def _(d):
            for c in range(CHUNK):
                v = [rows_buf[c * H + h, pl.ds(d, L)] for h in range(H)]
                while len(v) > 1:
                    nxt = [v[i] + v[i + 1] for i in range(0, len(v) - 1, 2)]
                    if len(v) % 2:
                        nxt.append(v[-1])
                    v = nxt
                pooled_buf[row_base + c, pl.ds(d, L)] = v[0]

    for b in range(NBUF):
        start(b, b)

    @pl.loop(0, NCHUNK, step=2 * NBUF)
    def _(i):
        for half in range(2):
            pooled_b, osem = pooled[half], osems[half]

            @pl.when(i > 0)
            def _():
                pltpu.make_async_copy(pooled_b, out_slice(i + half * NBUF), osem).wait()

            for b in range(NBUF):
                chunk = i + half * NBUF + b
                wait(chunk, b)
                reduce_chunk(rows[b], pooled_b, b * CHUNK)

                nxt = chunk + NBUF

                @pl.when(nxt < NCHUNK)
                def _():
                    start(nxt, b)

            pltpu.async_copy(pooled_b, out_slice(i + half * NBUF), osem)

    for half in range(2):
        pltpu.make_async_copy(pooled[half], out_slice(0), osems[half]).wait()


BLK = 4096


def _head_body(pooled_ref, w1_ref, b1_ref, w2_ref, b2_ref, out_ref):
    p = pooled_ref[...]
    w1 = w1_ref[...] * (1.0 / H)
    h = jnp.dot(p, w1, preferred_element_type=jnp.float32) + b1_ref[...]
    h = jnp.maximum(h, 0.0)
    out = jnp.sum(h * w2_ref[...], axis=1) + b2_ref[0, 0]
    out_ref[...] = out


_head = pl.pallas_call(
    _head_body,
    grid=(B // BLK,),
    in_specs=[
        pl.BlockSpec((BLK, D), lambda i: (i, 0)),
        pl.BlockSpec((D, HD), lambda i: (0, 0)),
        pl.BlockSpec((1, HD), lambda i: (0, 0)),
        pl.BlockSpec((1, HD), lambda i: (0, 0)),
        pl.BlockSpec((1, 1), lambda i: (0, 0), memory_space=pltpu.SMEM),
    ],
    out_specs=pl.BlockSpec((BLK,), lambda i: (i,)),
    out_shape=jax.ShapeDtypeStruct((B,), jnp.float32),
)


def kernel(x, embed_table, W1, b1, W2, b2):
    pooled = _sc_pool(x, embed_table)
    return _head(pooled, W1, b1.reshape(1, HD), W2.reshape(1, HD),
                 b2.reshape(1, 1))


# speedup vs baseline: 1.2937x; 1.0234x over previous
import dataclasses
import functools

import jax
import jax.numpy as jnp
from jax import lax
from jax.experimental import pallas as pl
from jax.experimental.pallas import tpu as pltpu
from jax.experimental.pallas import tpu_sc as plsc

VOCAB = 100000
D = 128
HD = 128
B = 16384
H = 20

NC = 2
NS = 16
NW = NC * NS
L = 16

B_PER_W = B // NW
CHUNK = 4
IDX_PER_CHUNK = CHUNK * H
NCHUNK = B_PER_W // CHUNK
NBUF = 8
GROUP_ROWS = NBUF * CHUNK

_mesh = plsc.VectorSubcoreMesh(core_axis_name="c", subcore_axis_name="s")

_cp = pltpu.CompilerParams()
if "needs_layout_passes" in pltpu.CompilerParams.__dataclass_fields__:
    _cp = dataclasses.replace(_cp, needs_layout_passes=False)

_scratch = (
    [pltpu.VMEM((B_PER_W // 8, H), jnp.int32)] * 2
    + [pltpu.SemaphoreType.DMA] * 2
    + [pltpu.VMEM((B_PER_W * H,), jnp.int32)]
    + [pltpu.VMEM((IDX_PER_CHUNK, D), jnp.float32)] * NBUF
    + [pltpu.VMEM((GROUP_ROWS, D), jnp.float32)] * 2
    + [pltpu.SemaphoreType.DMA] * NBUF
    + [pltpu.SemaphoreType.DMA] * 2
)


@functools.partial(
    pl.kernel,
    out_type=jax.ShapeDtypeStruct((B, D), jnp.float32),
    mesh=_mesh,
    scratch_types=_scratch,
    compiler_params=_cp,
)
def _sc_pool(x_hbm, table_hbm, out_hbm, stage0, stage1, ssem0, ssem1, idx_v, *refs):
    rows = refs[:NBUF]
    pooled = refs[NBUF:NBUF + 2]
    gsems = refs[NBUF + 2:2 * NBUF + 2]
    osems = refs[2 * NBUF + 2:]
    stages = ((stage0, ssem0), (stage1, ssem1))

    wid = lax.axis_index("s") * NC + lax.axis_index("c")
    out_base = wid * B_PER_W

    lane = lax.broadcasted_iota(jnp.int32, (L,), 0)
    patt = []
    for k in range(5):
        pos = lane + (k * L)
        patt.append((pos // H, pos % H))

    NPIECE = 8
    PIECE = B_PER_W // NPIECE

    def stage_start(piece, sb):
        buf, sem = stages[sb]
        pltpu.async_copy(x_hbm.at[pl.ds(out_base + piece * PIECE, PIECE)], buf, sem)

    def stage_wait(piece, sb):
        buf, sem = stages[sb]
        pltpu.make_async_copy(
            x_hbm.at[pl.ds(out_base + piece * PIECE, PIECE)], buf, sem).wait()

    def repack(piece, sb):
        buf = stages[sb][0]

        @pl.loop(0, PIECE // CHUNK)
        def _(sg):
            row_off = sg * CHUNK
            for k in range(5):
                r, c = patt[k]
                vals = plsc.load_gather(buf, [r + row_off, c])
                idx_v[pl.ds(piece * PIECE * H + sg * IDX_PER_CHUNK + k * L, L)] = vals

    def idx_slice(chunk):
        return idx_v.at[pl.ds(chunk * IDX_PER_CHUNK, IDX_PER_CHUNK)]

    def start(chunk, b):
        pltpu.async_copy(table_hbm.at[idx_slice(chunk)], rows[b], gsems[b])

    def wait(chunk, b):
        pltpu.make_async_copy(table_hbm.at[idx_slice(chunk)], rows[b], gsems[b]).wait()

    def out_slice(chunk):
        return out_hbm.at[pl.ds(out_base + chunk * CHUNK, GROUP_ROWS)]

    def reduce_chunk(rows_buf, pooled_buf, row_base):
        @pl.loop(0, D, step=L)
        def _(d):
            for c in range(CHUNK):
                v = [rows_buf[c * H + h, pl.ds(d, L)] for h in range(H)]
                while len(v) > 1:
                    nxt = [v[i] + v[i + 1] for i in range(0, len(v) - 1, 2)]
                    if len(v) % 2:
                        nxt.append(v[-1])
                    v = nxt
                pooled_buf[row_base + c, pl.ds(d, L)] = v[0]

    stage_start(0, 0)
    stage_start(1, 1)
    stage_wait(0, 0)
    repack(0, 0)
    for b in range(NBUF):
        start(b, b)
    stage_start(2, 0)
    for p in range(1, NPIECE):
        sb = p % 2
        stage_wait(p, sb)
        repack(p, sb)
        if p + 2 < NPIECE:
            stage_start(p + 2, sb)

    @pl.loop(0, NCHUNK, step=2 * NBUF)
    def _(i):
        for half in range(2):
            pooled_b, osem = pooled[half], osems[half]

            @pl.when(i > 0)
            def _():
                pltpu.make_async_copy(pooled_b, out_slice(i + half * NBUF), osem).wait()

            for b in range(NBUF):
                chunk = i + half * NBUF + b
                wait(chunk, b)
                reduce_chunk(rows[b], pooled_b, b * CHUNK)

                nxt = chunk + NBUF

                @pl.when(nxt < NCHUNK)
                def _():
                    start(nxt, b)

            pltpu.async_copy(pooled_b, out_slice(i + half * NBUF), osem)

    for half in range(2):
        pltpu.make_async_copy(pooled[half], out_slice(0), osems[half]).wait()


BLK = 4096


def _head_body(pooled_ref, w1_ref, b1_ref, w2_ref, b2_ref, out_ref):
    p = pooled_ref[...]
    w1 = w1_ref[...] * (1.0 / H)
    h = jnp.dot(p, w1, preferred_element_type=jnp.float32) + b1_ref[...]
    h = jnp.maximum(h, 0.0)
    out = jnp.sum(h * w2_ref[...], axis=1) + b2_ref[0, 0]
    out_ref[...] = out


_head = pl.pallas_call(
    _head_body,
    grid=(B // BLK,),
    in_specs=[
        pl.BlockSpec((BLK, D), lambda i: (i, 0)),
        pl.BlockSpec((D, HD), lambda i: (0, 0)),
        pl.BlockSpec((1, HD), lambda i: (0, 0)),
        pl.BlockSpec((1, HD), lambda i: (0, 0)),
        pl.BlockSpec((1, 1), lambda i: (0, 0), memory_space=pltpu.SMEM),
    ],
    out_specs=pl.BlockSpec((BLK,), lambda i: (i,)),
    out_shape=jax.ShapeDtypeStruct((B,), jnp.float32),
)


def kernel(x, embed_table, W1, b1, W2, b2):
    pooled = _sc_pool(x, embed_table)
    return _head(pooled, W1, b1.reshape(1, HD), W2.reshape(1, HD),
                 b2.reshape(1, 1))
